# Initial kernel scaffold; baseline (speedup 1.0000x reference)
import jax
import jax.numpy as jnp
from jax.experimental import pallas as pl


def kernel(P, H, I, N, M, W_rows, W_cols, W_vals, Wdx_rows, Wdx_cols, Wdx_vals, Wdy_rows, Wdy_cols, Wdy_vals):
    raise NotImplementedError("write your pallas kernel here")


# speedup vs baseline: 215.2690x; 215.2690x over previous
import jax
import jax.numpy as jnp
from jax import lax
from jax.experimental import pallas as pl
from jax.experimental.pallas import tpu as pltpu
from jax.experimental.pallas import tpu_sc as plsc

_XP = 128
_YP = 128
_T = 50
_L = _XP * _YP
_LT = _L * _T
_NNZ = 4 * _LT
_PIX = 10.0 / _YP / 20.0
_SCALE = 2.0
_RI = 1.33

_NC = 2
_NS = 16
_NTILES = _NC * _NS
_PER_TILE = _NNZ // _NTILES
_C = 2048
_NROW = _C // 128
_NWIN = _PER_TILE // _C
_OUT_SLICE = _LT // _NS
_ZC = 12800


def _spmv_body(h_hbm, r0, c0, v0, r1, c1, v1, r2, c2, v2, out_hbm,
               h_v, rows_v, cols_v, vals_v, contrib_v, zero_v, acc_sh, sem):
    core = lax.axis_index("c")
    sub = lax.axis_index("s")

    def zfill(i, carry):
        zero_v[pl.ds(i * 16, 16)] = jnp.zeros((16,), jnp.float32)
        return carry

    lax.fori_loop(0, _ZC // 16, zfill, 0)
    pltpu.sync_copy(h_hbm, h_v)

    for m, (rh, ch, vh) in enumerate(((r0, c0, v0), (r1, c1, v1), (r2, c2, v2))):
        for z in range(_OUT_SLICE // _ZC):
            pltpu.sync_copy(zero_v,
                            acc_sh.at[pl.ds(sub * _OUT_SLICE + z * _ZC, _ZC)])
        plsc.subcore_barrier()

        tile_base = (core * _NS + sub) * _PER_TILE

        def wbody(w, carry):
            nnz0 = pl.multiple_of(tile_base + w * _C, _C)
            row0 = pl.multiple_of(nnz0 // 128, _NROW)
            d1 = pltpu.async_copy(rh.at[pl.ds(row0, _NROW)], rows_v, sem)
            d2 = pltpu.async_copy(ch.at[pl.ds(nnz0, _C)], cols_v, sem)
            d3 = pltpu.async_copy(vh.at[pl.ds(nnz0, _C)], vals_v, sem)
            d1.wait()
            d2.wait()
            d3.wait()

            def gbody(i, gcarry):
                off = i * 16
                idx = cols_v[pl.ds(off, 16)]
                hv = plsc.load_gather(h_v, [idx])
                contrib_v[pl.ds(off, 16)] = hv * vals_v[pl.ds(off, 16)]
                return gcarry

            lax.fori_loop(0, _C // 16, gbody, 0)

            descs = [
                pltpu.async_copy(contrib_v.at[pl.ds(j * 128, 128)],
                                 acc_sh.at[rows_v.at[j]], sem, add=True)
                for j in range(_NROW)
            ]
            for dd in descs:
                dd.wait()
            return carry

        lax.fori_loop(0, _NWIN, wbody, 0)
        plsc.subcore_barrier()
        pltpu.sync_copy(acc_sh.at[pl.ds(sub * _OUT_SLICE, _OUT_SLICE)],
                        out_hbm.at[m, core, pl.ds(sub * _OUT_SLICE, _OUT_SLICE)])
        plsc.subcore_barrier()


def _make_spmv():
    mesh = plsc.VectorSubcoreMesh(core_axis_name="c", subcore_axis_name="s")
    return pl.kernel(
        _spmv_body,
        out_type=jax.ShapeDtypeStruct((3, _NC, _LT), jnp.float32),
        mesh=mesh,
        compiler_params=pltpu.CompilerParams(needs_layout_passes=False),
        scratch_types=[
            pltpu.VMEM((_L,), jnp.float32),
            pltpu.VMEM((_NROW, 128), jnp.int32),
            pltpu.VMEM((_C,), jnp.int32),
            pltpu.VMEM((_C,), jnp.float32),
            pltpu.VMEM((_C,), jnp.float32),
            pltpu.VMEM((_ZC,), jnp.float32),
            pltpu.VMEM_SHARED((_LT,), jnp.float32),
            pltpu.SemaphoreType.DMA,
        ],
    )


def _dense_body(wh0, wh1, dxa, dxb, dya, dyb, i0, i1, i2, mm, p0, p1, p2, out):
    t = pl.program_id(0)
    wh = wh0[0] + wh1[0]
    k = 1.0 / _PIX / _SCALE
    gx = (dxa[0] + dxb[0]) * k
    gy = (dya[0] + dyb[0]) * k
    a0 = i0[0]
    a1 = i1[0]
    a2 = i2[0]
    s0 = a0 * wh
    s1 = a1 * wh
    s2 = a2 * wh
    sn = jnp.sqrt(s0 * s0 + s1 * s1 + s2 * s2)
    e0 = (0.0 - s0) / sn
    e1 = (0.0 - s1) / sn
    e2 = (0.0 - s2) / sn
    n2 = -a0 * gx - a1 * gy - wh
    nn = jnp.sqrt(gx * gx + gy * gy + n2 * n2)
    b0 = gx / nn
    b1 = gy / nn
    b2 = n2 / nn
    cs = b0 * e0 + b1 * e1 + b2 * e2
    ir = 1.0 / _RI
    fac = ir * cs - jnp.sqrt(1.0 - ir * ir * (1.0 - cs * cs))
    r0 = fac * b0 - ir * e0
    r1 = fac * b1 - ir * e1
    r2 = fac * b2 - ir * e2
    d0 = p0[...] - s0
    d1 = p1[...] - s1
    d2 = p2[...] - s2
    dr = d0 * r0 + d1 * r1 + d2 * r2
    q0 = d0 - dr * r0
    q1 = d1 - dr * r1
    q2 = d2 - dr * r2
    bs = jnp.sum(mm[0] * (q0 * q0 + q1 * q1 + q2 * q2))

    @pl.when(t == 0)
    def _():
        out[0, 0] = 0.0

    out[0, 0] += bs


def _dense_call(wh0, wh1, dxa, dxb, dya, dyb, i0, i1, i2, mm, p0, p1, p2):
    stream = pl.BlockSpec((1, 128, 128), lambda t: (t, 0, 0))
    fixed = pl.BlockSpec((128, 128), lambda t: (0, 0))
    return pl.pallas_call(
        _dense_body,
        grid=(_T,),
        in_specs=[stream] * 10 + [fixed] * 3,
        out_specs=pl.BlockSpec(memory_space=pltpu.SMEM),
        out_shape=jax.ShapeDtypeStruct((1, 1), jnp.float32),
    )(wh0, wh1, dxa, dxb, dya, dyb, i0, i1, i2, mm, p0, p1, p2)


def kernel(P, H, I, N, M, W_rows, W_cols, W_vals,
           Wdx_rows, Wdx_cols, Wdx_vals, Wdy_rows, Wdy_cols, Wdy_vals):
    r0 = W_rows.reshape(_NNZ // 128, 128)
    r1 = Wdx_rows.reshape(_NNZ // 128, 128)
    r2 = Wdy_rows.reshape(_NNZ // 128, 128)
    parts = _make_spmv()(H, r0, W_cols, W_vals, r1, Wdx_cols, Wdx_vals,
                         r2, Wdy_cols, Wdy_vals)
    pr = parts.reshape(3, _NC, _T, 128, 128)
    i3 = I.reshape(3, _T, 128, 128)
    mm = M.reshape(_T, 128, 128)
    p3 = P.reshape(3, 128, 128)
    D = _dense_call(pr[0, 0], pr[0, 1], pr[1, 0], pr[1, 1], pr[2, 0], pr[2, 1],
                    i3[0], i3[1], i3[2], mm, p3[0], p3[1], p3[2])
    return D.reshape(())


# speedup vs baseline: 220.6221x; 1.0249x over previous
import jax
import jax.numpy as jnp
from jax import lax
from jax.experimental import pallas as pl
from jax.experimental.pallas import tpu as pltpu
from jax.experimental.pallas import tpu_sc as plsc

_XP = 128
_YP = 128
_T = 50
_L = _XP * _YP
_LT = _L * _T
_NNZ = 4 * _LT
_PIX = 10.0 / _YP / 20.0
_SCALE = 2.0
_RI = 1.33

_NC = 2
_NS = 16
_NTILES = _NC * _NS
_PER_TILE = _NNZ // _NTILES
_C = 2048
_NROW = _C // 128
_NWIN = _PER_TILE // _C
_OUT_SLICE = _LT // _NS
_ZC = 12800


def _spmv_body(h_hbm, r0, c0, v0, r1, c1, v1, r2, c2, v2, out_hbm,
               h_v, rows_a, rows_b, cols_v, vals_v, contrib_a, contrib_b,
               zero_v, acc_sh, sem, sem_sc):
    core = lax.axis_index("c")
    sub = lax.axis_index("s")

    def zfill(i, carry):
        zero_v[pl.ds(i * 16, 16)] = jnp.zeros((16,), jnp.float32)
        return carry

    lax.fori_loop(0, _ZC // 16, zfill, 0)
    pltpu.sync_copy(h_hbm, h_v)

    for m, (rh, ch, vh) in enumerate(((r0, c0, v0), (r1, c1, v1), (r2, c2, v2))):
        for z in range(_OUT_SLICE // _ZC):
            pltpu.sync_copy(zero_v,
                            acc_sh.at[pl.ds(sub * _OUT_SLICE + z * _ZC, _ZC)])
        plsc.subcore_barrier()

        tile_base = (core * _NS + sub) * _PER_TILE

        def issue_loads(w, rows_buf):
            nnz0 = pl.multiple_of(tile_base + w * _C, _C)
            row0 = pl.multiple_of(nnz0 // 128, _NROW)
            d1 = pltpu.async_copy(rh.at[pl.ds(row0, _NROW)], rows_buf, sem)
            d2 = pltpu.async_copy(ch.at[pl.ds(nnz0, _C)], cols_v, sem)
            d3 = pltpu.async_copy(vh.at[pl.ds(nnz0, _C)], vals_v, sem)
            return d1, d2, d3

        def wait_all(descs):
            for d in descs:
                d.wait()

        def compute(contrib_buf):
            def gbody(i, gcarry):
                off = i * 16
                idx = cols_v[pl.ds(off, 16)]
                hv = plsc.load_gather(h_v, [idx])
                contrib_buf[pl.ds(off, 16)] = hv * vals_v[pl.ds(off, 16)]
                return gcarry

            lax.fori_loop(0, _C // 16, gbody, 0, unroll=8)

        def issue_scatter(rows_buf, contrib_buf):
            return [
                pltpu.async_copy(contrib_buf.at[pl.ds(j * 128, 128)],
                                 acc_sh.at[rows_buf.at[j]], sem_sc, add=True)
                for j in range(_NROW)
            ]

        def drain_one_window():
            pltpu.make_async_copy(vh.at[pl.ds(0, _C)], contrib_a, sem_sc).wait()

        wait_all(issue_loads(0, rows_a))

        def kbody(k, carry):
            w0 = k * 2
            compute(contrib_a)
            s0 = issue_scatter(rows_a, contrib_a)

            @pl.when(k > 0)
            def _():
                drain_one_window()

            wait_all(issue_loads(w0 + 1, rows_b))
            compute(contrib_b)
            issue_scatter(rows_b, contrib_b)
            wait_all(s0)
            nxt = jnp.minimum(w0 + 2, _NWIN - 1)
            wait_all(issue_loads(nxt, rows_a))
            return carry

        lax.fori_loop(0, _NWIN // 2, kbody, 0)
        drain_one_window()
        plsc.subcore_barrier()
        pltpu.sync_copy(acc_sh.at[pl.ds(sub * _OUT_SLICE, _OUT_SLICE)],
                        out_hbm.at[m, core, pl.ds(sub * _OUT_SLICE, _OUT_SLICE)])
        plsc.subcore_barrier()


def _make_spmv():
    mesh = plsc.VectorSubcoreMesh(core_axis_name="c", subcore_axis_name="s")
    return pl.kernel(
        _spmv_body,
        out_type=jax.ShapeDtypeStruct((3, _NC, _LT), jnp.float32),
        mesh=mesh,
        compiler_params=pltpu.CompilerParams(needs_layout_passes=False),
        scratch_types=[
            pltpu.VMEM((_L,), jnp.float32),
            pltpu.VMEM((_NROW, 128), jnp.int32),
            pltpu.VMEM((_NROW, 128), jnp.int32),
            pltpu.VMEM((_C,), jnp.int32),
            pltpu.VMEM((_C,), jnp.float32),
            pltpu.VMEM((_C,), jnp.float32),
            pltpu.VMEM((_C,), jnp.float32),
            pltpu.VMEM((_ZC,), jnp.float32),
            pltpu.VMEM_SHARED((_LT,), jnp.float32),
            pltpu.SemaphoreType.DMA,
            pltpu.SemaphoreType.DMA,
        ],
    )


def _dense_body(wh0, wh1, dxa, dxb, dya, dyb, i0, i1, i2, mm, p0, p1, p2, out):
    t = pl.program_id(0)
    wh = wh0[0] + wh1[0]
    k = 1.0 / _PIX / _SCALE
    gx = (dxa[0] + dxb[0]) * k
    gy = (dya[0] + dyb[0]) * k
    a0 = i0[0]
    a1 = i1[0]
    a2 = i2[0]
    s0 = a0 * wh
    s1 = a1 * wh
    s2 = a2 * wh
    sn = jnp.sqrt(s0 * s0 + s1 * s1 + s2 * s2)
    e0 = (0.0 - s0) / sn
    e1 = (0.0 - s1) / sn
    e2 = (0.0 - s2) / sn
    n2 = -a0 * gx - a1 * gy - wh
    nn = jnp.sqrt(gx * gx + gy * gy + n2 * n2)
    b0 = gx / nn
    b1 = gy / nn
    b2 = n2 / nn
    cs = b0 * e0 + b1 * e1 + b2 * e2
    ir = 1.0 / _RI
    fac = ir * cs - jnp.sqrt(1.0 - ir * ir * (1.0 - cs * cs))
    r0 = fac * b0 - ir * e0
    r1 = fac * b1 - ir * e1
    r2 = fac * b2 - ir * e2
    d0 = p0[...] - s0
    d1 = p1[...] - s1
    d2 = p2[...] - s2
    dr = d0 * r0 + d1 * r1 + d2 * r2
    q0 = d0 - dr * r0
    q1 = d1 - dr * r1
    q2 = d2 - dr * r2
    bs = jnp.sum(mm[0] * (q0 * q0 + q1 * q1 + q2 * q2))

    @pl.when(t == 0)
    def _():
        out[0, 0] = 0.0

    out[0, 0] += bs


def _dense_call(wh0, wh1, dxa, dxb, dya, dyb, i0, i1, i2, mm, p0, p1, p2):
    stream = pl.BlockSpec((1, 128, 128), lambda t: (t, 0, 0))
    fixed = pl.BlockSpec((128, 128), lambda t: (0, 0))
    return pl.pallas_call(
        _dense_body,
        grid=(_T,),
        in_specs=[stream] * 10 + [fixed] * 3,
        out_specs=pl.BlockSpec(memory_space=pltpu.SMEM),
        out_shape=jax.ShapeDtypeStruct((1, 1), jnp.float32),
    )(wh0, wh1, dxa, dxb, dya, dyb, i0, i1, i2, mm, p0, p1, p2)


def kernel(P, H, I, N, M, W_rows, W_cols, W_vals,
           Wdx_rows, Wdx_cols, Wdx_vals, Wdy_rows, Wdy_cols, Wdy_vals):
    r0 = W_rows.reshape(_NNZ // 128, 128)
    r1 = Wdx_rows.reshape(_NNZ // 128, 128)
    r2 = Wdy_rows.reshape(_NNZ // 128, 128)
    parts = _make_spmv()(H, r0, W_cols, W_vals, r1, Wdx_cols, Wdx_vals,
                         r2, Wdy_cols, Wdy_vals)
    pr = parts.reshape(3, _NC, _T, 128, 128)
    i3 = I.reshape(3, _T, 128, 128)
    mm = M.reshape(_T, 128, 128)
    p3 = P.reshape(3, 128, 128)
    D = _dense_call(pr[0, 0], pr[0, 1], pr[1, 0], pr[1, 1], pr[2, 0], pr[2, 1],
                    i3[0], i3[1], i3[2], mm, p3[0], p3[1], p3[2])
    return D.reshape(())
